# baseline (speedup 1.0000x reference)
import functools

import jax
import jax.numpy as jnp
from jax import lax
from jax.experimental import pallas as pl
from jax.experimental.pallas import tpu as pltpu
from jax.experimental.pallas import tpu_sc as plsc

_L = 16


@functools.cache
def _make_gather(BT, H, D):
    info = plsc.get_sparse_core_info()
    NC, NS = info.num_cores, info.num_subcores
    NW = NC * NS
    assert BT % (NW * 128) == 0 and D % 8 == 0 and H % 2 == 0
    W = BT // NW
    E1, B1 = D // 8, BT // 128
    WB = W // 128
    mesh = plsc.VectorSubcoreMesh(core_axis_name="c", subcore_axis_name="s")

    @functools.partial(
        pl.kernel,
        mesh=mesh,
        out_type=jax.ShapeDtypeStruct((H, E1, B1, 8, 128), jnp.float32),
        scratch_types=[
            pltpu.VMEM((H, W), jnp.int32),
            pltpu.VMEM((2, W, D), jnp.float32),
            pltpu.VMEM((2, E1, WB, 8, 128), jnp.float32),
            pltpu.SemaphoreType.DMA((2,)),
            pltpu.SemaphoreType.DMA((2,)),
        ],
        compiler_params=pltpu.CompilerParams(
            use_tc_tiling_on_sc=False, needs_layout_passes=False),
    )
    def k(idx_hbm, table_hbm, y_hbm, idxT_v, rows_v, rowsT_v, gsem, ssem):
        wid = lax.axis_index("s") * NC + lax.axis_index("c")
        pltpu.sync_copy(idx_hbm.at[:, pl.ds(wid * W, W)], idxT_v)
        lanes = lax.iota(jnp.int32, _L)

        def gather_cp(h, b):
            return pltpu.make_async_copy(
                table_hbm.at[idxT_v.at[h]], rows_v.at[b], gsem.at[b])

        def store_cp(h, b):
            return pltpu.make_async_copy(
                rowsT_v.at[b], y_hbm.at[h, :, pl.ds(wid * WB, WB)], ssem.at[b])

        def transpose_rows(b):
            def e_body(e, carry):
                r, e0 = e // 8, e % 8
                e_vec = jnp.full((_L,), e, jnp.int32)

                def j_body(j, carry2):
                    vec = plsc.load_gather(
                        rows_v.at[b], [j * _L + lanes, e_vec])
                    rowsT_v[b, r, j // 8, e0, pl.ds((j % 8) * _L, _L)] = vec
                    return carry2
                lax.fori_loop(0, W // _L, j_body, 0, unroll=16)
                return carry
            lax.fori_loop(0, D, e_body, 0)

        gather_cp(0, 0).start()

        def half(t, b):
            h = 2 * t + b
            gather_cp(h, b).wait()
            if b == 0:
                gather_cp(h + 1, 1 - b).start()
            else:
                @pl.when(t < H // 2 - 1)
                def _():
                    gather_cp(h + 1, 1 - b).start()

            @pl.when(t > 0)
            def _():
                store_cp(h - 2, b).wait()
            transpose_rows(b)
            store_cp(h, b).start()

        def body(t, carry):
            half(t, 0)
            half(t, 1)
            return carry

        lax.fori_loop(0, H // 2, body, 0)
        store_cp(H - 2, 0).wait()
        store_cp(H - 1, 1).wait()

    return k


def kernel(input_tensor, table):
    bt, h = input_tensor.shape
    v, d = table.shape
    y = _make_gather(bt, h, d)(input_tensor.T, table)
    return y.transpose(2, 4, 0, 1, 3).reshape(bt, h, d)


# speedup vs baseline: 1.6007x; 1.6007x over previous
import functools

import jax
import jax.numpy as jnp
from jax import lax
from jax.experimental import pallas as pl
from jax.experimental.pallas import tpu as pltpu
from jax.experimental.pallas import tpu_sc as plsc

_L = 16


@functools.cache
def _make_gather(BT, H, D):
    info = plsc.get_sparse_core_info()
    NC, NS = info.num_cores, info.num_subcores
    NW = NC * NS
    assert BT % (NW * 128) == 0 and D % 8 == 0 and H % 2 == 0
    W = BT // NW
    E1, B1 = D // 8, BT // 128
    WB = W // 128
    mesh = plsc.VectorSubcoreMesh(core_axis_name="c", subcore_axis_name="s")

    @functools.partial(
        pl.kernel,
        mesh=mesh,
        out_type=jax.ShapeDtypeStruct((H, E1, B1, 8, 128), jnp.float32),
        scratch_types=[
            pltpu.VMEM((H, W), jnp.int32),
            pltpu.VMEM((2, W, D), jnp.float32),
            pltpu.VMEM((2, E1, WB, 8, 129), jnp.float32),
            pltpu.SemaphoreType.DMA((2,)),
            pltpu.SemaphoreType.DMA((2,)),
        ],
        compiler_params=pltpu.CompilerParams(
            use_tc_tiling_on_sc=False, needs_layout_passes=False),
    )
    def k(idx_hbm, table_hbm, y_hbm, idxT_v, rows_v, rowsT_v, gsem, ssem):
        wid = lax.axis_index("s") * NC + lax.axis_index("c")
        pltpu.sync_copy(idx_hbm.at[:, pl.ds(wid * W, W)], idxT_v)
        lanes = lax.iota(jnp.int32, _L)

        def gather_cp(h, b):
            return pltpu.make_async_copy(
                table_hbm.at[idxT_v.at[h]], rows_v.at[b], gsem.at[b])

        def store_cp(h, b):
            return pltpu.make_async_copy(
                rowsT_v.at[b].at[:, :, :, pl.ds(0, 128)],
                y_hbm.at[h, :, pl.ds(wid * WB, WB)], ssem.at[b])

        half_idx = [((2 * hf + lanes // 8), (lanes % 8)) for hf in range(D // _L)]

        def transpose_rows(b):
            def c_body(c, carry):
                c_vec = jnp.full((_L,), c, jnp.int32)

                def b_body(b0, carry2):
                    b_vec = jnp.full((_L,), b0, jnp.int32)
                    row = c * 128 + b0
                    for hf, (r_vec, e_vec) in enumerate(half_idx):
                        vec = rows_v[b, row, pl.ds(hf * _L, _L)]
                        plsc.store_scatter(
                            rowsT_v.at[b], [r_vec, c_vec, e_vec, b_vec], vec)
                    return carry2
                lax.fori_loop(0, 128, b_body, 0, unroll=8)
                return carry
            lax.fori_loop(0, WB, c_body, 0)

        gather_cp(0, 0).start()

        def half(t, b):
            h = 2 * t + b
            gather_cp(h, b).wait()
            if b == 0:
                gather_cp(h + 1, 1 - b).start()
            else:
                @pl.when(t < H // 2 - 1)
                def _():
                    gather_cp(h + 1, 1 - b).start()

            @pl.when(t > 0)
            def _():
                store_cp(h - 2, b).wait()
            transpose_rows(b)
            store_cp(h, b).start()

        def body(t, carry):
            half(t, 0)
            half(t, 1)
            return carry

        lax.fori_loop(0, H // 2, body, 0)
        store_cp(H - 2, 0).wait()
        store_cp(H - 1, 1).wait()

    return k


def kernel(input_tensor, table):
    bt, h = input_tensor.shape
    v, d = table.shape
    y = _make_gather(bt, h, d)(input_tensor.T, table)
    return y.transpose(2, 4, 0, 1, 3).reshape(bt, h, d)
